# baseline (speedup 1.0000x reference)
import jax
import jax.numpy as jnp
import numpy as np
from jax import lax
from jax.experimental import pallas as pl
from jax.experimental.pallas import tpu as pltpu
from jax.experimental.pallas import tpu_sc as plsc
from functools import partial

_N = 10000
_DEG = 16
_HID = 64
_HARM = 5
_EPAD = 176128
_OFFCAP = _EPAD - _N

_TM_E = 2048
_TM_N = 2000

_NC = 2
_NS = 16
_NW = _NC * _NS
_CHUNK = 128
_PER_W = _EPAD // _NW
_NCHUNK = _PER_W // _CHUNK
_NPT = _N // _NS


def _full(i):
    return (0, 0)


def _row(i):
    return (i, 0)


def _mlp3_kernel(x_ref, w1_ref, b1_ref, w2_ref, b2_ref, w3_ref, b3_ref, o_ref):
    h = jnp.maximum(jnp.dot(x_ref[...], w1_ref[...],
                            preferred_element_type=jnp.float32) + b1_ref[...], 0.0)
    h = jnp.maximum(jnp.dot(h, w2_ref[...],
                            preferred_element_type=jnp.float32) + b2_ref[...], 0.0)
    o_ref[...] = jnp.dot(h, w3_ref[...],
                         preferred_element_type=jnp.float32) + b3_ref[...]


def _edge_block_kernel(e_ref, s_ref, r_ref, m_ref,
                       wa_ref, wb_ref, wc_ref, b1_ref,
                       w2_ref, b2_ref, w3_ref, b3_ref,
                       enew_ref, emask_ref):
    e = e_ref[...]
    h = jnp.dot(e, wa_ref[...], preferred_element_type=jnp.float32)
    h += jnp.dot(s_ref[...], wb_ref[...], preferred_element_type=jnp.float32)
    h += jnp.dot(r_ref[...], wc_ref[...], preferred_element_type=jnp.float32)
    h = jnp.maximum(h + b1_ref[...], 0.0)
    h = jnp.maximum(jnp.dot(h, w2_ref[...],
                            preferred_element_type=jnp.float32) + b2_ref[...], 0.0)
    enew = e + jnp.dot(h, w3_ref[...],
                       preferred_element_type=jnp.float32) + b3_ref[...]
    enew_ref[...] = enew
    emask_ref[...] = enew * m_ref[...]


def _node_block_kernel(n_ref, a0_ref, a1_ref, d_ref,
                       wa_ref, wb_ref, b1_ref,
                       w2_ref, b2_ref, w3_ref, b3_ref, o_ref):
    n = n_ref[...]
    a = (a0_ref[...] + a1_ref[...]) / jnp.maximum(d_ref[...], 1.0)
    h = jnp.dot(n, wa_ref[...], preferred_element_type=jnp.float32)
    h += jnp.dot(a, wb_ref[...], preferred_element_type=jnp.float32)
    h = jnp.maximum(h + b1_ref[...], 0.0)
    h = jnp.maximum(jnp.dot(h, w2_ref[...],
                            preferred_element_type=jnp.float32) + b2_ref[...], 0.0)
    o_ref[...] = n + jnp.dot(h, w3_ref[...],
                             preferred_element_type=jnp.float32) + b3_ref[...]


def _kan_kernel(x_ref, w_ref, b_ref, o_ref):
    x = x_ref[...]
    cols = []
    for i in range(3):
        xi = x[:, i:i + 1]
        cols.append(jnp.ones_like(xi))
        for k in range(1, _HARM + 1):
            cols.append(jnp.sin(k * xi))
            cols.append(jnp.cos(k * xi))
    basis = jnp.concatenate(cols, axis=-1)
    o_ref[...] = jnp.dot(basis, w_ref[...],
                         preferred_element_type=jnp.float32) + b_ref[...]


def _dec_kernel(n_ref, x2_ref, w1_ref, b1_ref, w2_ref, b2_ref,
                w3_ref, b3_ref, o_ref):
    h = jnp.maximum(jnp.dot(n_ref[...], w1_ref[...],
                            preferred_element_type=jnp.float32) + b1_ref[...], 0.0)
    h = jnp.maximum(jnp.dot(h, w2_ref[...],
                            preferred_element_type=jnp.float32) + b2_ref[...], 0.0)
    o_ref[...] = x2_ref[...] + jnp.dot(h, w3_ref[...],
                                       preferred_element_type=jnp.float32) + b3_ref[...]


def _sc_mesh():
    return plsc.VectorSubcoreMesh(core_axis_name="c", subcore_axis_name="s")


_ROWS_W = 314
_GRP = _ROWS_W // 2
_WORDS = _N // 4
_FULLC = _WORDS // 16
_FLUSH = 8192
_ECAP = _FLUSH + _N + 16 + 16


def _extract_body(adj_hbm, base_hbm, out_s, out_r, adjbuf, sbuf, rbuf, basev):
    wid = lax.axis_index("s") * _NC + lax.axis_index("c")
    wstart = wid * _ROWS_W
    lanes = jax.lax.iota(jnp.int32, 16)
    iota4 = lanes * 4

    pltpu.sync_copy(base_hbm, basev)
    b0 = basev[pl.ds(0, 16)]
    b1 = basev[pl.ds(16, 16)]
    bsel = jnp.where(wid < 16, b0, b1)
    base = jnp.sum(jnp.where(lanes == wid % 16, bsel, 0))

    def emit(w, c0, lane_ok, r, cu):
        for k in range(4):
            bk = (w >> (8 * k)) & 0xFF
            mk = (bk != 0) & lane_ok
            pos = plsc.cumsum(jnp.where(mk, 1, 0))
            idx = cu + pos - 1
            plsc.store_scatter(rbuf, [idx], iota4 + (c0 + k), mask=mk)
            plsc.store_scatter(sbuf, [idx], jnp.full((16,), 0, jnp.int32) + r,
                               mask=mk)
            cu = cu + jnp.sum(jnp.where(mk, 1, 0))
        return cu

    def process_row(g, r, cu):
        def chunk_body(c, cu):
            w = adjbuf[pl.ds(g * _WORDS + c * 16, 16)]
            nz = jnp.sum(jnp.where(w != 0, 1, 0))
            return lax.cond(nz > 0,
                            lambda cu: emit(w, c * 64, lanes >= 0, r, cu),
                            lambda cu: cu, cu)
        cu = lax.fori_loop(0, _FULLC, chunk_body, cu)
        w = adjbuf[pl.ds(g * _WORDS + _WORDS - 16, 16)]
        nz = jnp.sum(jnp.where(w != 0, 1, 0))
        return lax.cond(nz > 0,
                        lambda cu: emit(w, (_WORDS - 16) * 4, lanes >= 12, r, cu),
                        lambda cu: cu, cu)

    def do_flush(cu, fl):
        off = pl.multiple_of(base + fl, 128)
        pltpu.sync_copy(sbuf.at[pl.ds(0, _FLUSH)],
                        out_s.at[pl.ds(off, _FLUSH)])
        pltpu.sync_copy(rbuf.at[pl.ds(0, _FLUSH)],
                        out_r.at[pl.ds(off, _FLUSH)])

        def move(t, carry):
            sv = sbuf[pl.ds(_FLUSH + t * 16, 16)]
            rv = rbuf[pl.ds(_FLUSH + t * 16, 16)]
            sbuf[pl.ds(t * 16, 16)] = sv
            rbuf[pl.ds(t * 16, 16)] = rv
            return carry

        lax.fori_loop(0, (_ECAP - _FLUSH) // 16, move, 0)
        return cu - _FLUSH, fl + _FLUSH

    def group_body(gi, carry):
        cu, fl = carry
        r0 = wstart + 2 * gi

        @pl.when(r0 < _N)
        def _():
            pltpu.sync_copy(
                adj_hbm.at[pl.ds(pl.multiple_of(r0 * _WORDS, 8), 2 * _WORDS)],
                adjbuf)

        for g in range(2):
            r = r0 + g
            cu = lax.cond(r < _N,
                          lambda cu: process_row(g, r, cu),
                          lambda cu: cu, cu)
        return lax.cond(cu >= _FLUSH, do_flush, lambda cu, fl: (cu, fl), cu, fl)

    cu, fl = lax.fori_loop(0, _GRP, group_body, (jnp.int32(0), jnp.int32(0)))

    def tail_flush(t, carry):
        cu, fl = carry

        @pl.when(t * 128 < cu)
        def _():
            off = pl.multiple_of(base + fl + t * 128, 128)
            pltpu.sync_copy(sbuf.at[pl.ds(t * 128, 128)],
                            out_s.at[pl.ds(off, 128)])
            pltpu.sync_copy(rbuf.at[pl.ds(t * 128, 128)],
                            out_r.at[pl.ds(off, 128)])
        return carry

    lax.fori_loop(0, _FLUSH // 128, tail_flush, (cu, fl))


def _sc_extract(adj32, bases):
    k = partial(
        pl.kernel,
        mesh=_sc_mesh(),
        compiler_params=pltpu.CompilerParams(use_tc_tiling_on_sc=False,
                                             needs_layout_passes=False),
        out_type=(jax.ShapeDtypeStruct((_OFFCAP,), jnp.int32),
                  jax.ShapeDtypeStruct((_OFFCAP,), jnp.int32)),
        scratch_types=[
            pltpu.VMEM((2 * _WORDS,), jnp.int32),
            pltpu.VMEM((_ECAP,), jnp.int32),
            pltpu.VMEM((_ECAP,), jnp.int32),
            pltpu.VMEM((32,), jnp.int32),
        ],
    )(_extract_body)
    return k(adj32, bases)


def _gather2_body(node_hbm, s_hbm, r_hbm, out_s, out_r,
                  sidx, ridx, srows, rrows, sem1, sem2):
    wid = lax.axis_index("s") * _NC + lax.axis_index("c")
    base = wid * _PER_W

    def issue(t, b):
        off = base + t * _CHUNK
        pltpu.sync_copy(s_hbm.at[pl.ds(off, _CHUNK)], sidx.at[b])
        pltpu.sync_copy(r_hbm.at[pl.ds(off, _CHUNK)], ridx.at[b])
        pltpu.async_copy(node_hbm.at[sidx.at[b]], srows.at[b], sem1)
        pltpu.async_copy(node_hbm.at[ridx.at[b]], rrows.at[b], sem2)

    def drain_store(t, b):
        off = base + t * _CHUNK
        pltpu.make_async_copy(node_hbm.at[sidx.at[b]], srows.at[b], sem1).wait()
        pltpu.make_async_copy(node_hbm.at[ridx.at[b]], rrows.at[b], sem2).wait()
        pltpu.sync_copy(srows.at[b], out_s.at[pl.ds(off, _CHUNK)])
        pltpu.sync_copy(rrows.at[b], out_r.at[pl.ds(off, _CHUNK)])

    issue(0, 0)

    def body(t, carry):
        issue(t, t % 2)
        drain_store(t - 1, (t - 1) % 2)
        return carry

    lax.fori_loop(1, _NCHUNK, body, 0)
    drain_store(_NCHUNK - 1, (_NCHUNK - 1) % 2)


def _sc_gather2(node, senders, receivers):
    k = partial(
        pl.kernel,
        mesh=_sc_mesh(),
        compiler_params=pltpu.CompilerParams(use_tc_tiling_on_sc=False),
        out_type=(jax.ShapeDtypeStruct((_EPAD, _HID), jnp.float32),
                  jax.ShapeDtypeStruct((_EPAD, _HID), jnp.float32)),
        scratch_types=[
            pltpu.VMEM((2, _CHUNK), jnp.int32),
            pltpu.VMEM((2, _CHUNK), jnp.int32),
            pltpu.VMEM((2, _CHUNK, _HID), jnp.float32),
            pltpu.VMEM((2, _CHUNK, _HID), jnp.float32),
            pltpu.SemaphoreType.DMA,
            pltpu.SemaphoreType.DMA,
        ],
    )(_gather2_body)
    return k(node, senders, receivers)


def _scatter_add_body(emask_hbm, ridx_hbm, zeros_hbm, out_hbm,
                      idx_v, rows_v, sem, shared):
    sid = lax.axis_index("s")
    scid = lax.axis_index("c")
    wid = sid * _NC + scid
    base = wid * _PER_W
    pltpu.sync_copy(zeros_hbm, shared.at[pl.ds(sid * _NPT, _NPT)])
    plsc.subcore_barrier()

    def body(t, carry):
        off = base + t * _CHUNK
        pltpu.sync_copy(ridx_hbm.at[pl.ds(off, _CHUNK)], idx_v)
        pltpu.async_copy(emask_hbm.at[pl.ds(off, _CHUNK)], rows_v, sem).wait()
        pltpu.sync_copy(rows_v, shared.at[idx_v], add=True)
        return carry

    lax.fori_loop(0, _NCHUNK, body, 0)
    plsc.subcore_barrier()
    pltpu.sync_copy(shared.at[pl.ds(sid * _NPT, _NPT)],
                    out_hbm.at[scid, pl.ds(sid * _NPT, _NPT)])


def _sc_scatter_add(emask, receivers, zeros_stripe):
    k = partial(
        pl.kernel,
        mesh=_sc_mesh(),
        compiler_params=pltpu.CompilerParams(use_tc_tiling_on_sc=False),
        out_type=jax.ShapeDtypeStruct((_NC, _N, _HID), jnp.float32),
        scratch_types=[
            pltpu.VMEM((_CHUNK,), jnp.int32),
            pltpu.VMEM((_CHUNK, _HID), jnp.float32),
            pltpu.SemaphoreType.DMA,
            pltpu.VMEM_SHARED((_N, _HID), jnp.float32),
        ],
    )(_scatter_add_body)
    return k(emask, receivers, zeros_stripe)


def _wspec(shape):
    return pl.BlockSpec(shape, _full)


def _rspec(tm, d):
    return pl.BlockSpec((tm, d), _row)


def _call_mlp3(x, p, tm):
    rows = x.shape[0]
    din = x.shape[1]
    dout = p[2][0].shape[1]
    grid = (rows // tm,)
    return pl.pallas_call(
        _mlp3_kernel,
        grid=grid,
        in_specs=[_rspec(tm, din),
                  _wspec(p[0][0].shape), _wspec((1, p[0][1].shape[0])),
                  _wspec(p[1][0].shape), _wspec((1, p[1][1].shape[0])),
                  _wspec(p[2][0].shape), _wspec((1, p[2][1].shape[0]))],
        out_specs=_rspec(tm, dout),
        out_shape=jax.ShapeDtypeStruct((rows, dout), jnp.float32),
    )(x, p[0][0], p[0][1][None], p[1][0], p[1][1][None], p[2][0], p[2][1][None])


def _call_edge_block(edge, sf, rf, vmask, bp):
    rows = edge.shape[0]
    grid = (rows // _TM_E,)
    w1 = bp[0][0]
    wa, wb, wc = w1[:_HID], w1[_HID:2 * _HID], w1[2 * _HID:]
    return pl.pallas_call(
        _edge_block_kernel,
        grid=grid,
        in_specs=[_rspec(_TM_E, _HID)] * 3 + [_rspec(_TM_E, 1)] +
                 [_wspec((_HID, _HID))] * 3 + [_wspec((1, _HID))] +
                 [_wspec((_HID, _HID)), _wspec((1, _HID)),
                  _wspec((_HID, _HID)), _wspec((1, _HID))],
        out_specs=(_rspec(_TM_E, _HID), _rspec(_TM_E, _HID)),
        out_shape=(jax.ShapeDtypeStruct((rows, _HID), jnp.float32),
                   jax.ShapeDtypeStruct((rows, _HID), jnp.float32)),
    )(edge, sf, rf, vmask, wa, wb, wc, bp[0][1][None],
      bp[1][0], bp[1][1][None], bp[2][0], bp[2][1][None])


def _call_node_block(node, agg0, agg1, deg, bp):
    rows = node.shape[0]
    grid = (rows // _TM_N,)
    w1 = bp[0][0]
    wa, wb = w1[:_HID], w1[_HID:]
    return pl.pallas_call(
        _node_block_kernel,
        grid=grid,
        in_specs=[_rspec(_TM_N, _HID), _rspec(_TM_N, _HID),
                  _rspec(_TM_N, _HID), _rspec(_TM_N, 1)] +
                 [_wspec((_HID, _HID))] * 2 + [_wspec((1, _HID))] +
                 [_wspec((_HID, _HID)), _wspec((1, _HID)),
                  _wspec((_HID, _HID)), _wspec((1, _HID))],
        out_specs=_rspec(_TM_N, _HID),
        out_shape=jax.ShapeDtypeStruct((rows, _HID), jnp.float32),
    )(node, agg0, agg1, deg, wa, wb, bp[0][1][None],
      bp[1][0], bp[1][1][None], bp[2][0], bp[2][1][None])


def _call_kan(node_x, kan):
    rows = node_x.shape[0]
    w = jnp.concatenate([kan[i][0] for i in range(3)], axis=0)
    b = (kan[0][1] + kan[1][1] + kan[2][1])[None]
    grid = (rows // _TM_N,)
    return pl.pallas_call(
        _kan_kernel,
        grid=grid,
        in_specs=[_rspec(_TM_N, 3), _wspec((33, _HID)), _wspec((1, _HID))],
        out_specs=_rspec(_TM_N, _HID),
        out_shape=jax.ShapeDtypeStruct((rows, _HID), jnp.float32),
    )(node_x, w, b)


def _call_dec(node, x2, p):
    rows = node.shape[0]
    grid = (rows // _TM_N,)
    return pl.pallas_call(
        _dec_kernel,
        grid=grid,
        in_specs=[_rspec(_TM_N, _HID), _rspec(_TM_N, 2),
                  _wspec((_HID, _HID)), _wspec((1, _HID)),
                  _wspec((_HID, _HID)), _wspec((1, _HID)),
                  _wspec((_HID, 2)), _wspec((1, 2))],
        out_specs=_rspec(_TM_N, 2),
        out_shape=jax.ShapeDtypeStruct((rows, 2), jnp.float32),
    )(node, x2, p[0][0], p[0][1][None], p[1][0], p[1][1][None],
      p[2][0], p[2][1][None])


def kernel(node_x, coords, adj, params):
    a = adj.astype(bool)
    n = a.shape[0]
    idx = jnp.arange(n)
    offdiag = (a & (idx[:, None] != idx[None, :])).astype(jnp.uint8)
    adj32 = lax.bitcast_convert_type(
        offdiag.reshape(_N * _WORDS, 4), jnp.int32)
    rowcnt = jnp.sum(offdiag, axis=1, dtype=jnp.int32)
    wcnt = jnp.sum(
        jnp.concatenate([rowcnt, jnp.zeros((_NW * _ROWS_W - _N,), jnp.int32)]
                        ).reshape(_NW, _ROWS_W), axis=1)
    reserved = (wcnt + 127) & ~127
    bases = jnp.concatenate([jnp.zeros((1,), jnp.int32),
                             jnp.cumsum(reserved)[:-1]])
    s_off, r_off = _sc_extract(adj32, bases)
    senders = jnp.concatenate([jnp.clip(s_off, 0, _N - 1), idx])
    receivers = jnp.concatenate([jnp.clip(r_off, 0, _N - 1), idx])
    j = jnp.arange(_OFFCAP, dtype=jnp.int32)[:, None]
    valid_off = jnp.any((j >= bases[None, :]) &
                        (j < (bases + wcnt)[None, :]), axis=1)
    vmask = jnp.concatenate([valid_off.astype(jnp.float32),
                             jnp.ones((_N,), jnp.float32)])[:, None]

    nx = node_x[0]
    node = _call_kan(nx, params["kan"])

    delta = coords[senders] - coords[receivers]
    dist = jnp.linalg.norm(delta, axis=-1, keepdims=True)
    edge_feat = jnp.concatenate([delta, dist], axis=-1)
    edge = _call_mlp3(edge_feat, params["edge_enc"], _TM_E)

    deg = (jnp.sum(offdiag, axis=0, dtype=jnp.float32) + 1.0)[:, None]
    zeros_stripe = jnp.zeros((_NPT, _HID), jnp.float32)
    for bp in params["blocks"]:
        sf, rf = _sc_gather2(node, senders, receivers)
        edge, emask = _call_edge_block(edge, sf, rf, vmask, bp["edge"])
        agg = _sc_scatter_add(emask, receivers, zeros_stripe)
        node = _call_node_block(node, agg[0], agg[1], deg, bp["node"])

    out = _call_dec(node, nx[:, :2], params["dec"])
    return out[None]


# speedup vs baseline: 1.7825x; 1.7825x over previous
import jax
import jax.numpy as jnp
import numpy as np
from jax import lax
from jax.experimental import pallas as pl
from jax.experimental.pallas import tpu as pltpu
from jax.experimental.pallas import tpu_sc as plsc
from functools import partial

_N = 10000
_DEG = 16
_HID = 64
_HARM = 5
_EPAD = 176128
_OFFCAP = _EPAD - _N

_TM_E = 2048
_TM_N = 2000

_NC = 2
_NS = 16
_NW = _NC * _NS
_CHUNK = 128
_PER_W = _EPAD // _NW
_NCHUNK = _PER_W // _CHUNK
_NPT = _N // _NS


def _full(i):
    return (0, 0)


def _row(i):
    return (i, 0)


def _mlp3_kernel(x_ref, w1_ref, b1_ref, w2_ref, b2_ref, w3_ref, b3_ref, o_ref):
    h = jnp.maximum(jnp.dot(x_ref[...], w1_ref[...],
                            preferred_element_type=jnp.float32) + b1_ref[...], 0.0)
    h = jnp.maximum(jnp.dot(h, w2_ref[...],
                            preferred_element_type=jnp.float32) + b2_ref[...], 0.0)
    o_ref[...] = jnp.dot(h, w3_ref[...],
                         preferred_element_type=jnp.float32) + b3_ref[...]


def _edge_block_kernel(e_ref, s_ref, r_ref, m_ref,
                       wa_ref, wb_ref, wc_ref, b1_ref,
                       w2_ref, b2_ref, w3_ref, b3_ref,
                       enew_ref, emask_ref):
    e = e_ref[...]
    h = jnp.dot(e, wa_ref[...], preferred_element_type=jnp.float32)
    h += jnp.dot(s_ref[...], wb_ref[...], preferred_element_type=jnp.float32)
    h += jnp.dot(r_ref[...], wc_ref[...], preferred_element_type=jnp.float32)
    h = jnp.maximum(h + b1_ref[...], 0.0)
    h = jnp.maximum(jnp.dot(h, w2_ref[...],
                            preferred_element_type=jnp.float32) + b2_ref[...], 0.0)
    enew = e + jnp.dot(h, w3_ref[...],
                       preferred_element_type=jnp.float32) + b3_ref[...]
    enew_ref[...] = enew
    emask_ref[...] = enew * m_ref[...]


def _node_block_kernel(n_ref, a0_ref, a1_ref, d_ref,
                       wa_ref, wb_ref, b1_ref,
                       w2_ref, b2_ref, w3_ref, b3_ref, o_ref):
    n = n_ref[...]
    a = (a0_ref[...] + a1_ref[...]) / jnp.maximum(d_ref[...], 1.0)
    h = jnp.dot(n, wa_ref[...], preferred_element_type=jnp.float32)
    h += jnp.dot(a, wb_ref[...], preferred_element_type=jnp.float32)
    h = jnp.maximum(h + b1_ref[...], 0.0)
    h = jnp.maximum(jnp.dot(h, w2_ref[...],
                            preferred_element_type=jnp.float32) + b2_ref[...], 0.0)
    o_ref[...] = n + jnp.dot(h, w3_ref[...],
                             preferred_element_type=jnp.float32) + b3_ref[...]


def _kan_kernel(x_ref, w_ref, b_ref, o_ref):
    x = x_ref[...]
    cols = []
    for i in range(3):
        xi = x[:, i:i + 1]
        cols.append(jnp.ones_like(xi))
        for k in range(1, _HARM + 1):
            cols.append(jnp.sin(k * xi))
            cols.append(jnp.cos(k * xi))
    basis = jnp.concatenate(cols, axis=-1)
    o_ref[...] = jnp.dot(basis, w_ref[...],
                         preferred_element_type=jnp.float32) + b_ref[...]


def _dec_kernel(n_ref, x2_ref, w1_ref, b1_ref, w2_ref, b2_ref,
                w3_ref, b3_ref, o_ref):
    h = jnp.maximum(jnp.dot(n_ref[...], w1_ref[...],
                            preferred_element_type=jnp.float32) + b1_ref[...], 0.0)
    h = jnp.maximum(jnp.dot(h, w2_ref[...],
                            preferred_element_type=jnp.float32) + b2_ref[...], 0.0)
    o_ref[...] = x2_ref[...] + jnp.dot(h, w3_ref[...],
                                       preferred_element_type=jnp.float32) + b3_ref[...]


def _sc_mesh():
    return plsc.VectorSubcoreMesh(core_axis_name="c", subcore_axis_name="s")


_ROWS_W = 314
_GRP = _ROWS_W // 2
_WORDS = _N // 4
_FULLC = _WORDS // 16
_FLUSH = 8192
_ECAP = _FLUSH + _N + 16 + 16


def _extract_body(adj_hbm, base_hbm, out_s, out_r, adjbuf, sbuf, rbuf, basev):
    wid = lax.axis_index("s") * _NC + lax.axis_index("c")
    wstart = wid * _ROWS_W
    lanes = jax.lax.iota(jnp.int32, 16)
    iota4 = lanes * 4

    pltpu.sync_copy(base_hbm, basev)
    b0 = basev[pl.ds(0, 16)]
    b1 = basev[pl.ds(16, 16)]
    bsel = jnp.where(wid < 16, b0, b1)
    base = jnp.sum(jnp.where(lanes == wid % 16, bsel, 0))

    def emit(w, c0, lane_ok, r, cu):
        for k in range(4):
            bk = (w >> (8 * k)) & 0xFF
            mk = (bk != 0) & lane_ok
            pos = plsc.cumsum(jnp.where(mk, 1, 0))
            idx = cu + pos - 1
            plsc.store_scatter(rbuf, [idx], iota4 + (c0 + k), mask=mk)
            plsc.store_scatter(sbuf, [idx], jnp.full((16,), 0, jnp.int32) + r,
                               mask=mk)
            cu = cu + jnp.sum(jnp.where(mk, 1, 0))
        return cu

    def process_row(g, r, cu):
        def chunk_body(c, cu):
            w = adjbuf[g, pl.ds(c * 16, 16)]
            nz = jnp.sum(jnp.where(w != 0, 1, 0))
            return lax.cond(nz > 0,
                            lambda cu: emit(w, c * 64, lanes >= 0, r, cu),
                            lambda cu: cu, cu)
        cu = lax.fori_loop(0, _FULLC, chunk_body, cu)
        w = adjbuf[g, pl.ds(_WORDS - 16, 16)]
        nz = jnp.sum(jnp.where(w != 0, 1, 0))
        return lax.cond(nz > 0,
                        lambda cu: emit(w, (_WORDS - 16) * 4, lanes >= 12, r, cu),
                        lambda cu: cu, cu)

    def do_flush(cu, fl):
        off = pl.multiple_of(base + fl, 128)
        pltpu.sync_copy(sbuf.at[pl.ds(0, _FLUSH)],
                        out_s.at[pl.ds(off, _FLUSH)])
        pltpu.sync_copy(rbuf.at[pl.ds(0, _FLUSH)],
                        out_r.at[pl.ds(off, _FLUSH)])

        def move(t, carry):
            sv = sbuf[pl.ds(_FLUSH + t * 16, 16)]
            rv = rbuf[pl.ds(_FLUSH + t * 16, 16)]
            sbuf[pl.ds(t * 16, 16)] = sv
            rbuf[pl.ds(t * 16, 16)] = rv
            return carry

        lax.fori_loop(0, (_ECAP - _FLUSH) // 16, move, 0)
        return cu - _FLUSH, fl + _FLUSH

    def group_body(gi, carry):
        cu, fl = carry
        r0 = wstart + 2 * gi

        @pl.when(r0 < _N)
        def _():
            pltpu.sync_copy(adj_hbm.at[pl.ds(r0, 2)], adjbuf)

        for g in range(2):
            r = r0 + g
            cu = lax.cond(r < _N,
                          lambda cu: process_row(g, r, cu),
                          lambda cu: cu, cu)
        return lax.cond(cu >= _FLUSH, do_flush, lambda cu, fl: (cu, fl), cu, fl)

    cu, fl = lax.fori_loop(0, _GRP, group_body, (jnp.int32(0), jnp.int32(0)))

    def tail_flush(t, carry):
        cu, fl = carry

        @pl.when(t * 128 < cu)
        def _():
            off = pl.multiple_of(base + fl + t * 128, 128)
            pltpu.sync_copy(sbuf.at[pl.ds(t * 128, 128)],
                            out_s.at[pl.ds(off, 128)])
            pltpu.sync_copy(rbuf.at[pl.ds(t * 128, 128)],
                            out_r.at[pl.ds(off, 128)])
        return carry

    lax.fori_loop(0, _FLUSH // 128, tail_flush, (cu, fl))


def _sc_extract(adj32, bases):
    k = partial(
        pl.kernel,
        mesh=_sc_mesh(),
        compiler_params=pltpu.CompilerParams(use_tc_tiling_on_sc=False,
                                             needs_layout_passes=False),
        out_type=(jax.ShapeDtypeStruct((_OFFCAP,), jnp.int32),
                  jax.ShapeDtypeStruct((_OFFCAP,), jnp.int32)),
        scratch_types=[
            pltpu.VMEM((2, _WORDS), jnp.int32),
            pltpu.VMEM((_ECAP,), jnp.int32),
            pltpu.VMEM((_ECAP,), jnp.int32),
            pltpu.VMEM((32,), jnp.int32),
        ],
    )(_extract_body)
    return k(adj32, bases)


def _gather2_body(node_hbm, s_hbm, r_hbm, out_s, out_r,
                  sidx, ridx, srows, rrows, sem1, sem2):
    wid = lax.axis_index("s") * _NC + lax.axis_index("c")
    base = wid * _PER_W

    def issue(t, b):
        off = base + t * _CHUNK
        pltpu.sync_copy(s_hbm.at[pl.ds(off, _CHUNK)], sidx.at[b])
        pltpu.sync_copy(r_hbm.at[pl.ds(off, _CHUNK)], ridx.at[b])
        pltpu.async_copy(node_hbm.at[sidx.at[b]], srows.at[b], sem1)
        pltpu.async_copy(node_hbm.at[ridx.at[b]], rrows.at[b], sem2)

    def drain_store(t, b):
        off = base + t * _CHUNK
        pltpu.make_async_copy(node_hbm.at[sidx.at[b]], srows.at[b], sem1).wait()
        pltpu.make_async_copy(node_hbm.at[ridx.at[b]], rrows.at[b], sem2).wait()
        pltpu.sync_copy(srows.at[b], out_s.at[pl.ds(off, _CHUNK)])
        pltpu.sync_copy(rrows.at[b], out_r.at[pl.ds(off, _CHUNK)])

    issue(0, 0)

    def body(t, carry):
        issue(t, t % 2)
        drain_store(t - 1, (t - 1) % 2)
        return carry

    lax.fori_loop(1, _NCHUNK, body, 0)
    drain_store(_NCHUNK - 1, (_NCHUNK - 1) % 2)


def _sc_gather2(node, senders, receivers):
    k = partial(
        pl.kernel,
        mesh=_sc_mesh(),
        compiler_params=pltpu.CompilerParams(use_tc_tiling_on_sc=False),
        out_type=(jax.ShapeDtypeStruct((_EPAD, _HID), jnp.float32),
                  jax.ShapeDtypeStruct((_EPAD, _HID), jnp.float32)),
        scratch_types=[
            pltpu.VMEM((2, _CHUNK), jnp.int32),
            pltpu.VMEM((2, _CHUNK), jnp.int32),
            pltpu.VMEM((2, _CHUNK, _HID), jnp.float32),
            pltpu.VMEM((2, _CHUNK, _HID), jnp.float32),
            pltpu.SemaphoreType.DMA,
            pltpu.SemaphoreType.DMA,
        ],
    )(_gather2_body)
    return k(node, senders, receivers)


def _scatter_add_body(emask_hbm, ridx_hbm, zeros_hbm, out_hbm,
                      idx_v, rows_v, sem, shared):
    sid = lax.axis_index("s")
    scid = lax.axis_index("c")
    wid = sid * _NC + scid
    base = wid * _PER_W
    pltpu.sync_copy(zeros_hbm, shared.at[pl.ds(sid * _NPT, _NPT)])
    plsc.subcore_barrier()

    def body(t, carry):
        off = base + t * _CHUNK
        pltpu.sync_copy(ridx_hbm.at[pl.ds(off, _CHUNK)], idx_v)
        pltpu.async_copy(emask_hbm.at[pl.ds(off, _CHUNK)], rows_v, sem).wait()
        pltpu.sync_copy(rows_v, shared.at[idx_v], add=True)
        return carry

    lax.fori_loop(0, _NCHUNK, body, 0)
    plsc.subcore_barrier()
    pltpu.sync_copy(shared.at[pl.ds(sid * _NPT, _NPT)],
                    out_hbm.at[scid, pl.ds(sid * _NPT, _NPT)])


def _sc_scatter_add(emask, receivers, zeros_stripe):
    k = partial(
        pl.kernel,
        mesh=_sc_mesh(),
        compiler_params=pltpu.CompilerParams(use_tc_tiling_on_sc=False),
        out_type=jax.ShapeDtypeStruct((_NC, _N, _HID), jnp.float32),
        scratch_types=[
            pltpu.VMEM((_CHUNK,), jnp.int32),
            pltpu.VMEM((_CHUNK, _HID), jnp.float32),
            pltpu.SemaphoreType.DMA,
            pltpu.VMEM_SHARED((_N, _HID), jnp.float32),
        ],
    )(_scatter_add_body)
    return k(emask, receivers, zeros_stripe)


def _wspec(shape):
    return pl.BlockSpec(shape, _full)


def _rspec(tm, d):
    return pl.BlockSpec((tm, d), _row)


def _call_mlp3(x, p, tm):
    rows = x.shape[0]
    din = x.shape[1]
    dout = p[2][0].shape[1]
    grid = (rows // tm,)
    return pl.pallas_call(
        _mlp3_kernel,
        grid=grid,
        in_specs=[_rspec(tm, din),
                  _wspec(p[0][0].shape), _wspec((1, p[0][1].shape[0])),
                  _wspec(p[1][0].shape), _wspec((1, p[1][1].shape[0])),
                  _wspec(p[2][0].shape), _wspec((1, p[2][1].shape[0]))],
        out_specs=_rspec(tm, dout),
        out_shape=jax.ShapeDtypeStruct((rows, dout), jnp.float32),
    )(x, p[0][0], p[0][1][None], p[1][0], p[1][1][None], p[2][0], p[2][1][None])


def _call_edge_block(edge, sf, rf, vmask, bp):
    rows = edge.shape[0]
    grid = (rows // _TM_E,)
    w1 = bp[0][0]
    wa, wb, wc = w1[:_HID], w1[_HID:2 * _HID], w1[2 * _HID:]
    return pl.pallas_call(
        _edge_block_kernel,
        grid=grid,
        in_specs=[_rspec(_TM_E, _HID)] * 3 + [_rspec(_TM_E, 1)] +
                 [_wspec((_HID, _HID))] * 3 + [_wspec((1, _HID))] +
                 [_wspec((_HID, _HID)), _wspec((1, _HID)),
                  _wspec((_HID, _HID)), _wspec((1, _HID))],
        out_specs=(_rspec(_TM_E, _HID), _rspec(_TM_E, _HID)),
        out_shape=(jax.ShapeDtypeStruct((rows, _HID), jnp.float32),
                   jax.ShapeDtypeStruct((rows, _HID), jnp.float32)),
    )(edge, sf, rf, vmask, wa, wb, wc, bp[0][1][None],
      bp[1][0], bp[1][1][None], bp[2][0], bp[2][1][None])


def _call_node_block(node, agg0, agg1, deg, bp):
    rows = node.shape[0]
    grid = (rows // _TM_N,)
    w1 = bp[0][0]
    wa, wb = w1[:_HID], w1[_HID:]
    return pl.pallas_call(
        _node_block_kernel,
        grid=grid,
        in_specs=[_rspec(_TM_N, _HID), _rspec(_TM_N, _HID),
                  _rspec(_TM_N, _HID), _rspec(_TM_N, 1)] +
                 [_wspec((_HID, _HID))] * 2 + [_wspec((1, _HID))] +
                 [_wspec((_HID, _HID)), _wspec((1, _HID)),
                  _wspec((_HID, _HID)), _wspec((1, _HID))],
        out_specs=_rspec(_TM_N, _HID),
        out_shape=jax.ShapeDtypeStruct((rows, _HID), jnp.float32),
    )(node, agg0, agg1, deg, wa, wb, bp[0][1][None],
      bp[1][0], bp[1][1][None], bp[2][0], bp[2][1][None])


def _call_kan(node_x, kan):
    rows = node_x.shape[0]
    w = jnp.concatenate([kan[i][0] for i in range(3)], axis=0)
    b = (kan[0][1] + kan[1][1] + kan[2][1])[None]
    grid = (rows // _TM_N,)
    return pl.pallas_call(
        _kan_kernel,
        grid=grid,
        in_specs=[_rspec(_TM_N, 3), _wspec((33, _HID)), _wspec((1, _HID))],
        out_specs=_rspec(_TM_N, _HID),
        out_shape=jax.ShapeDtypeStruct((rows, _HID), jnp.float32),
    )(node_x, w, b)


def _call_dec(node, x2, p):
    rows = node.shape[0]
    grid = (rows // _TM_N,)
    return pl.pallas_call(
        _dec_kernel,
        grid=grid,
        in_specs=[_rspec(_TM_N, _HID), _rspec(_TM_N, 2),
                  _wspec((_HID, _HID)), _wspec((1, _HID)),
                  _wspec((_HID, _HID)), _wspec((1, _HID)),
                  _wspec((_HID, 2)), _wspec((1, 2))],
        out_specs=_rspec(_TM_N, 2),
        out_shape=jax.ShapeDtypeStruct((rows, 2), jnp.float32),
    )(node, x2, p[0][0], p[0][1][None], p[1][0], p[1][1][None],
      p[2][0], p[2][1][None])


def kernel(node_x, coords, adj, params):
    a = adj.astype(bool)
    n = a.shape[0]
    idx = jnp.arange(n)
    offdiag = (a & (idx[:, None] != idx[None, :])).astype(jnp.uint8)
    adj32 = lax.bitcast_convert_type(
        offdiag.reshape(_N, _WORDS, 4), jnp.int32)
    rowcnt = jnp.sum(offdiag, axis=1, dtype=jnp.int32)
    wcnt = jnp.sum(
        jnp.concatenate([rowcnt, jnp.zeros((_NW * _ROWS_W - _N,), jnp.int32)]
                        ).reshape(_NW, _ROWS_W), axis=1)
    reserved = (wcnt + 127) & ~127
    bases = jnp.concatenate([jnp.zeros((1,), jnp.int32),
                             jnp.cumsum(reserved)[:-1]])
    s_off, r_off = _sc_extract(adj32, bases)
    senders = jnp.concatenate([jnp.clip(s_off, 0, _N - 1), idx])
    receivers = jnp.concatenate([jnp.clip(r_off, 0, _N - 1), idx])
    j = jnp.arange(_OFFCAP, dtype=jnp.int32)[:, None]
    valid_off = jnp.any((j >= bases[None, :]) &
                        (j < (bases + wcnt)[None, :]), axis=1)
    vmask = jnp.concatenate([valid_off.astype(jnp.float32),
                             jnp.ones((_N,), jnp.float32)])[:, None]

    nx = node_x[0]
    node = _call_kan(nx, params["kan"])

    delta = coords[senders] - coords[receivers]
    dist = jnp.linalg.norm(delta, axis=-1, keepdims=True)
    edge_feat = jnp.concatenate([delta, dist], axis=-1)
    edge = _call_mlp3(edge_feat, params["edge_enc"], _TM_E)

    deg = (jnp.sum(offdiag, axis=0, dtype=jnp.float32) + 1.0)[:, None]
    zeros_stripe = jnp.zeros((_NPT, _HID), jnp.float32)
    for bp in params["blocks"]:
        sf, rf = _sc_gather2(node, senders, receivers)
        edge, emask = _call_edge_block(edge, sf, rf, vmask, bp["edge"])
        agg = _sc_scatter_add(emask, receivers, zeros_stripe)
        node = _call_node_block(node, agg[0], agg[1], deg, bp["node"])

    out = _call_dec(node, nx[:, :2], params["dec"])
    return out[None]
